# baseline (speedup 1.0000x reference)
import functools

import jax
import jax.numpy as jnp
from jax import lax
from jax.experimental import pallas as pl
from jax.experimental.pallas import tpu as pltpu
from jax.experimental.pallas import tpu_sc as plsc

N = 10000
E = 320000
D = 128
NC = 2
NS = 16
NW = NC * NS
EPT = E // NW
CHM = 112
NFULL = 89
TL = EPT - NFULL * CHM
NIT = 17

DBLK = 2000
NDBLK = EPT // DBLK


def _deg_body(src_hbm, dst_hbm, ew_hbm, out_hbm, bsrc, bdst, bew, accl):
    c = lax.axis_index("c")
    s = lax.axis_index("s")
    w = c * NS + s

    zero16 = jnp.zeros((16,), jnp.float32)
    one16 = jnp.ones((16,), jnp.float32)
    off16 = jnp.full((16,), N, jnp.int32)

    def zfill(i, _):
        accl[pl.ds(i * 16, 16)] = zero16
        return 0
    lax.fori_loop(0, (3 * N) // 16, zfill, 0)

    def blk(b, _):
        base = w * EPT + b * DBLK
        pltpu.sync_copy(src_hbm.at[pl.ds(base, DBLK)], bsrc)
        pltpu.sync_copy(dst_hbm.at[pl.ds(base, DBLK)], bdst)
        pltpu.sync_copy(ew_hbm.at[pl.ds(base, DBLK)], bew)

        def grp(i, _):
            sl = pl.ds(i * 16, 16)
            s16 = bsrc[sl]
            d16 = bdst[sl] + off16
            e16 = bew[sl]
            plsc.addupdate_scatter(accl, [s16], e16)
            plsc.addupdate_scatter(accl, [d16], e16)
            plsc.addupdate_scatter(accl, [d16 + off16], one16)
            return 0
        lax.fori_loop(0, DBLK // 16, grp, 0)
        return 0
    lax.fori_loop(0, NDBLK, blk, 0)

    pltpu.sync_copy(accl, out_hbm.at[pl.ds(w * 3 * N, 3 * N)])


def _mp_body(pk_hbm, h_hbm, out_hbm,
             pk0, pk1, d2_0, d2_1, d2_2, rows0, rows1, rows2, acc,
             semL0, semL1, semG0, semG1, semG2, semS0, semS1, semS2):
    c = lax.axis_index("c")
    s = lax.axis_index("s")
    w = c * NS + s

    I = [(pk0, semL0), (pk1, semL1)]
    R = [(rows0, d2_0, semG0, semS0), (rows1, d2_1, semG1, semS1),
         (rows2, d2_2, semG2, semS2)]

    zero16 = jnp.zeros((16,), jnp.float32)

    def zfill(i, _):
        for j in range(D // 16):
            rows0[i, pl.ds(j * 16, 16)] = zero16
        return 0
    lax.fori_loop(0, CHM, zfill, 0)

    @pl.when(s < 10)
    def _():
        for i in range(8):
            pltpu.async_copy(rows0,
                             acc.at[pl.ds(s * 1000 + i * CHM, CHM), :],
                             semL0)
        pltpu.async_copy(rows0.at[pl.ds(0, 104), :],
                         acc.at[pl.ds(s * 1000 + 896, 104), :], semL0)
        for i in range(8):
            pltpu.make_async_copy(
                rows0, acc.at[pl.ds(s * 1000 + i * CHM, CHM), :],
                semL0).wait()
        pltpu.make_async_copy(rows0.at[pl.ds(0, 104), :],
                              acc.at[pl.ds(s * 1000 + 896, 104), :],
                              semL0).wait()
    plsc.subcore_barrier()

    def loads_start(k, ib):
        base = (w * NFULL + k) * REC
        pltpu.async_copy(pk_hbm.at[pl.ds(base, REC)], ib[0], ib[1])

    def loads_wait(k, ib):
        base = (w * NFULL + k) * REC
        pltpu.make_async_copy(pk_hbm.at[pl.ds(base, REC)], ib[0],
                              ib[1]).wait()

    def idx_copy(ib, rb):
        pk_v = ib[0]
        d2_v = rb[1]
        for j in range(CHM // 16):
            d2_v[pl.ds(j * 16, 16)] = pk_v[pl.ds(CHM + j * 16, 16)]

    def gather_start(ib, rb):
        pltpu.async_copy(h_hbm.at[ib[0].at[pl.ds(0, CHM)]], rb[0], rb[2])

    def gather_wait(ib, rb):
        pltpu.make_async_copy(h_hbm.at[ib[0].at[pl.ds(0, CHM)]], rb[0],
                              rb[2]).wait()

    def scatter_start(rb):
        pltpu.async_copy(rb[0], acc.at[rb[1]], rb[3], add=True)

    def scatter_wait(rb):
        pltpu.make_async_copy(rb[0], acc.at[rb[1]], rb[3]).wait()

    def compute_chunk(rb, ib):
        rows = rb[0]
        pk_v = ib[0]

        def erow(g, _):
            sci16 = pk_v[pl.ds(2 * CHM + g * 16, 16)]
            scv16 = plsc.bitcast(sci16, jnp.float32)
            for l in range(16):
                e = g * 16 + l
                scv = jnp.full((16,), scv16[l], jnp.float32)
                for j in range(D // 16):
                    sl = pl.ds(j * 16, 16)
                    rows[e, sl] = rows[e, sl] * scv
            return 0
        lax.fori_loop(0, CHM // 16, erow, 0)

    def step(k, u, it, guard_first):
        icur = I[u % 2]
        inext = I[(u + 1) % 2]
        rcur = R[u % 3]
        rnext = R[(u + 1) % 3]
        loads_wait(k + 1, inext)
        if guard_first:
            @pl.when(it > 0)
            def _():
                scatter_wait(rnext)
        else:
            scatter_wait(rnext)
        idx_copy(inext, rnext)
        gather_start(inext, rnext)
        gather_wait(icur, rcur)
        compute_chunk(rcur, icur)
        scatter_start(rcur)
        loads_start(k + 2, icur)

    def tail_step(kk, u, has_next, has_next2):
        icur = I[u % 2]
        inext = I[(u + 1) % 2]
        rcur = R[u % 3]
        rnext = R[(u + 1) % 3]
        if has_next:
            loads_wait(kk + 1, inext)
        scatter_wait(rnext)
        if has_next:
            idx_copy(inext, rnext)
            gather_start(inext, rnext)
        gather_wait(icur, rcur)
        compute_chunk(rcur, icur)
        scatter_start(rcur)
        if has_next2:
            loads_start(kk + 2, icur)

    loads_start(0, I[0])
    loads_wait(0, I[0])
    idx_copy(I[0], R[0])
    gather_start(I[0], R[0])
    loads_start(1, I[1])

    def six(it, _):
        for u in range(6):
            step(6 * it + u, u, it, u < 2)
        return 0
    lax.fori_loop(0, NIT, six, 0)

    for kk in range(6 * NIT, NFULL):
        tail_step(kk, kk % 6, kk + 1 < NFULL, kk + 2 < NFULL)

    scatter_wait(R[(NFULL - 2) % 3])
    scatter_wait(R[(NFULL - 1) % 3])

    plsc.subcore_barrier()

    @pl.when(s < 10)
    def _():
        pltpu.sync_copy(acc.at[pl.ds(s * 1000, 1000), :],
                        out_hbm.at[pl.ds(c * N + s * 1000, 1000), :])


@functools.cache
def _sc_kernels():
    mesh = plsc.VectorSubcoreMesh(core_axis_name="c", subcore_axis_name="s",
                                  num_cores=NC, num_subcores=NS)
    deg_kernel = pl.kernel(
        _deg_body,
        out_type=jax.ShapeDtypeStruct((NW * 3 * N,), jnp.float32),
        mesh=mesh,
        compiler_params=pltpu.CompilerParams(needs_layout_passes=False),
        scratch_types=[
            pltpu.VMEM((DBLK,), jnp.int32),
            pltpu.VMEM((DBLK,), jnp.int32),
            pltpu.VMEM((DBLK,), jnp.float32),
            pltpu.VMEM((3 * N,), jnp.float32),
        ],
    )
    mp_kernel = pl.kernel(
        _mp_body,
        out_type=jax.ShapeDtypeStruct((NC * N, D), jnp.float32),
        mesh=mesh,
        compiler_params=pltpu.CompilerParams(needs_layout_passes=False),
        scratch_types=[
            pltpu.VMEM((REC,), jnp.int32),
            pltpu.VMEM((REC,), jnp.int32),
            pltpu.VMEM((CHM,), jnp.int32),
            pltpu.VMEM((CHM,), jnp.int32),
            pltpu.VMEM((CHM,), jnp.int32),
            pltpu.VMEM((CHM, D), jnp.float32),
            pltpu.VMEM((CHM, D), jnp.float32),
            pltpu.VMEM((CHM, D), jnp.float32),
            pltpu.VMEM_SHARED((N, D), jnp.float32),
            pltpu.SemaphoreType.DMA,
            pltpu.SemaphoreType.DMA,
            pltpu.SemaphoreType.DMA,
            pltpu.SemaphoreType.DMA,
            pltpu.SemaphoreType.DMA,
            pltpu.SemaphoreType.DMA,
            pltpu.SemaphoreType.DMA,
            pltpu.SemaphoreType.DMA,
        ],
    )
    return deg_kernel, mp_kernel


def _tcA_body(x_ref, w1_ref, degT_ref, eat_ref, we_ref, h_out, gate_out):
    dsrc = jnp.sum(degT_ref[0:N, :], axis=1, keepdims=True)
    a = lax.rsqrt(jnp.clip(dsrc, 1e-12))
    h = jnp.dot(x_ref[...], w1_ref[...], preferred_element_type=jnp.float32)
    h_out[...] = h * a
    gate_out[...] = jnp.sum(eat_ref[...] * we_ref[...], axis=0)


def _tcB_body(p_ref, degT_ref, b1_ref, g1_ref, be1_ref, w2_ref, h2_out):
    p = p_ref[...]
    agg = p[:N] + p[N:]
    ddst = jnp.sum(degT_ref[N:2 * N, :], axis=1, keepdims=True)
    bfac = lax.rsqrt(jnp.clip(ddst, 1e-12))
    agg = agg * bfac + b1_ref[...]
    mean = jnp.mean(agg, axis=0, keepdims=True)
    var = jnp.mean((agg - mean) ** 2, axis=0, keepdims=True)
    hb = (agg - mean) * lax.rsqrt(var + 1e-5) * g1_ref[...] + be1_ref[...]
    h2_out[...] = jnp.dot(hb, w2_ref[...],
                          preferred_element_type=jnp.float32)


def _tcC_body(p_ref, degT_ref, b2_ref, g2_ref, be2_ref, out_ref):
    p = p_ref[...]
    agg = p[:N] + p[N:]
    cnt = jnp.sum(degT_ref[2 * N:3 * N, :], axis=1, keepdims=True)
    agg = agg / jnp.clip(cnt, 1.0) + b2_ref[...]
    mean = jnp.mean(agg, axis=0, keepdims=True)
    var = jnp.mean((agg - mean) ** 2, axis=0, keepdims=True)
    hb = (agg - mean) * lax.rsqrt(var + 1e-5) * g2_ref[...] + be2_ref[...]
    out_ref[...] = jnp.where(hb > 0, hb, 0.1 * (jnp.exp(hb) - 1.0))


def _pack_records(src, dst, sca):
    pad = ((0, 0), (0, NFULL * CHM - EPT))
    s3 = jnp.pad(src.reshape(NW, EPT), pad).reshape(NW, NFULL, 1, CHM)
    d3 = jnp.pad(dst.reshape(NW, EPT), pad).reshape(NW, NFULL, 1, CHM)
    v3 = jnp.pad(lax.bitcast_convert_type(sca, jnp.int32).reshape(NW, EPT),
                 pad).reshape(NW, NFULL, 1, CHM)
    return jnp.concatenate([s3, d3, v3], axis=2).reshape(-1)


def kernel(x, edge_index, edge_attrs, W1, b1, g1, be1, we, W2, b2, g2, be2):
    src = edge_index[0]
    dst = edge_index[1]
    ew = edge_attrs[:, 1]
    eat = edge_attrs.T.reshape(4, E // D, D)

    deg_kernel, mp_kernel = _sc_kernels()
    degs = deg_kernel(src, dst, ew)
    degT = degs.reshape(NW, 3 * N).T

    h1s, gate2d = pl.pallas_call(
        _tcA_body,
        out_shape=[jax.ShapeDtypeStruct((N, D), jnp.float32),
                   jax.ShapeDtypeStruct((E // D, D), jnp.float32)],
    )(x, W1, degT, eat, we.reshape(4, 1, 1))
    gate = gate2d.reshape(E)

    p1 = mp_kernel(_pack_records(src, dst, ew), h1s)

    h2 = pl.pallas_call(
        _tcB_body,
        out_shape=jax.ShapeDtypeStruct((N, D), jnp.float32),
    )(p1, degT, b1.reshape(1, D), g1.reshape(1, D), be1.reshape(1, D), W2)

    p2 = mp_kernel(_pack_records(src, dst, gate), h2)

    out = pl.pallas_call(
        _tcC_body,
        out_shape=jax.ShapeDtypeStruct((N, D), jnp.float32),
    )(p2, degT, b2.reshape(1, D), g2.reshape(1, D), be2.reshape(1, D))
    return out


# speedup vs baseline: 1.5921x; 1.5921x over previous
import functools

import jax
import jax.numpy as jnp
from jax import lax
from jax.experimental import pallas as pl
from jax.experimental.pallas import tpu as pltpu
from jax.experimental.pallas import tpu_sc as plsc

N = 10000
E = 320000
D = 128
NC = 2
NS = 16
NW = NC * NS
EPT = E // NW
CHM = 112
NFULL = 89
TL = EPT - NFULL * CHM
NIT = 17

DBLK = 2000
NDBLK = EPT // DBLK


def _deg_body(src_hbm, dst_hbm, ew_hbm, out_hbm, bsrc, bdst, bew,
              bsrc2, bdst2, bew2, accl, semA, semB):
    c = lax.axis_index("c")
    s = lax.axis_index("s")
    w = c * NS + s

    zero16 = jnp.zeros((16,), jnp.float32)
    one16 = jnp.ones((16,), jnp.float32)
    off16 = jnp.full((16,), N, jnp.int32)

    def zfill(i, _):
        accl[pl.ds(i * 16, 16)] = zero16
        return 0
    lax.fori_loop(0, (3 * N) // 16, zfill, 0)

    B = [(bsrc, bdst, bew, semA), (bsrc2, bdst2, bew2, semB)]

    def dblk_start(b, bb):
        base = w * EPT + b * DBLK
        pltpu.async_copy(src_hbm.at[pl.ds(base, DBLK)], bb[0], bb[3])
        pltpu.async_copy(dst_hbm.at[pl.ds(base, DBLK)], bb[1], bb[3])
        pltpu.async_copy(ew_hbm.at[pl.ds(base, DBLK)], bb[2], bb[3])

    def dblk_wait(b, bb):
        base = w * EPT + b * DBLK
        pltpu.make_async_copy(src_hbm.at[pl.ds(base, DBLK)], bb[0],
                              bb[3]).wait()
        pltpu.make_async_copy(dst_hbm.at[pl.ds(base, DBLK)], bb[1],
                              bb[3]).wait()
        pltpu.make_async_copy(ew_hbm.at[pl.ds(base, DBLK)], bb[2],
                              bb[3]).wait()

    def dblk_compute(bb):
        def grp(i, _):
            sl = pl.ds(i * 16, 16)
            s16 = bb[0][sl]
            d16 = bb[1][sl] + off16
            e16 = bb[2][sl]
            plsc.addupdate_scatter(accl, [s16], e16)
            plsc.addupdate_scatter(accl, [d16], e16)
            plsc.addupdate_scatter(accl, [d16 + off16], one16)
            return 0
        lax.fori_loop(0, DBLK // 16, grp, 0)

    dblk_start(0, B[0])
    for b in range(NDBLK):
        if b + 1 < NDBLK:
            dblk_start(b + 1, B[(b + 1) % 2])
        dblk_wait(b, B[b % 2])
        dblk_compute(B[b % 2])

    pltpu.sync_copy(accl, out_hbm.at[pl.ds(w * 3 * N, 3 * N)])


def _mp_body(src_hbm, dst_hbm, sca_hbm, h_hbm, out_hbm,
             src0, dst0, sca0, src1, dst1, sca1,
             d2_0, d2_1, d2_2, rows0, rows1, rows2,
             srcT, dstT, scaT, acc,
             semL0, semL1, semG0, semG1, semG2, semS0, semS1, semS2,
             semT):
    c = lax.axis_index("c")
    s = lax.axis_index("s")
    w = c * NS + s
    tbase = w * EPT

    I = [(src0, dst0, sca0, semL0), (src1, dst1, sca1, semL1)]
    R = [(rows0, d2_0, semG0, semS0), (rows1, d2_1, semG1, semS1),
         (rows2, d2_2, semG2, semS2)]

    zero16 = jnp.zeros((16,), jnp.float32)

    def zfill(i, _):
        for j in range(D // 16):
            rows0[i, pl.ds(j * 16, 16)] = zero16
        return 0
    lax.fori_loop(0, CHM, zfill, 0)

    @pl.when(s < 10)
    def _():
        for i in range(8):
            pltpu.async_copy(rows0,
                             acc.at[pl.ds(s * 1000 + i * CHM, CHM), :],
                             semL0)
        pltpu.async_copy(rows0.at[pl.ds(0, 104), :],
                         acc.at[pl.ds(s * 1000 + 896, 104), :], semL0)
        for i in range(8):
            pltpu.make_async_copy(
                rows0, acc.at[pl.ds(s * 1000 + i * CHM, CHM), :],
                semL0).wait()
        pltpu.make_async_copy(rows0.at[pl.ds(0, 104), :],
                              acc.at[pl.ds(s * 1000 + 896, 104), :],
                              semL0).wait()
    plsc.subcore_barrier()

    def loads_start(k, ib):
        s_v, d_v, e_v, sem = ib
        base = tbase + k * CHM
        pltpu.async_copy(src_hbm.at[pl.ds(base, CHM)], s_v, sem)
        pltpu.async_copy(dst_hbm.at[pl.ds(base, CHM)], d_v, sem)
        pltpu.async_copy(sca_hbm.at[pl.ds(base, CHM)], e_v, sem)

    def loads_wait(k, ib):
        s_v, d_v, e_v, sem = ib
        base = tbase + k * CHM
        pltpu.make_async_copy(src_hbm.at[pl.ds(base, CHM)], s_v, sem).wait()
        pltpu.make_async_copy(dst_hbm.at[pl.ds(base, CHM)], d_v, sem).wait()
        pltpu.make_async_copy(sca_hbm.at[pl.ds(base, CHM)], e_v, sem).wait()

    def idx_copy(ib, rb):
        d_v = ib[1]
        d2_v = rb[1]
        for j in range(CHM // 16):
            sl = pl.ds(j * 16, 16)
            d2_v[sl] = d_v[sl]

    def gather_start(ib, rb):
        pltpu.async_copy(h_hbm.at[ib[0]], rb[0], rb[2])

    def gather_wait(ib, rb):
        pltpu.make_async_copy(h_hbm.at[ib[0]], rb[0], rb[2]).wait()

    def scatter_start(rb):
        pltpu.async_copy(rb[0], acc.at[rb[1]], rb[3], add=True)

    def scatter_wait(rb):
        pltpu.make_async_copy(rb[0], acc.at[rb[1]], rb[3]).wait()

    def compute_chunk(rb, ib):
        rows = rb[0]
        e_v = ib[2]

        def erow(g, _):
            scv16 = e_v[pl.ds(g * 16, 16)]
            for l in range(16):
                e = g * 16 + l
                scv = jnp.full((16,), scv16[l], jnp.float32)
                for j in range(D // 16):
                    sl = pl.ds(j * 16, 16)
                    rows[e, sl] = rows[e, sl] * scv
            return 0
        lax.fori_loop(0, CHM // 16, erow, 0)

    def step(k, u, it, guard_first):
        icur = I[u % 2]
        inext = I[(u + 1) % 2]
        rcur = R[u % 3]
        rnext = R[(u + 1) % 3]
        loads_wait(k + 1, inext)
        if guard_first:
            @pl.when(it > 0)
            def _():
                scatter_wait(rnext)
        else:
            scatter_wait(rnext)
        idx_copy(inext, rnext)
        gather_start(inext, rnext)
        gather_wait(icur, rcur)
        compute_chunk(rcur, icur)
        scatter_start(rcur)
        loads_start(k + 2, icur)

    def tail_step(kk, u, has_next, has_next2):
        icur = I[u % 2]
        inext = I[(u + 1) % 2]
        rcur = R[u % 3]
        rnext = R[(u + 1) % 3]
        if has_next:
            loads_wait(kk + 1, inext)
        scatter_wait(rnext)
        if has_next:
            idx_copy(inext, rnext)
            gather_start(inext, rnext)
        gather_wait(icur, rcur)
        compute_chunk(rcur, icur)
        scatter_start(rcur)
        if has_next2:
            loads_start(kk + 2, icur)

    loads_start(0, I[0])
    loads_wait(0, I[0])
    idx_copy(I[0], R[0])
    gather_start(I[0], R[0])
    loads_start(1, I[1])

    def six(it, _):
        for u in range(6):
            step(6 * it + u, u, it, u < 2)
        return 0
    lax.fori_loop(0, NIT, six, 0)

    for kk in range(6 * NIT, NFULL):
        tail_step(kk, kk % 6, kk + 1 < NFULL, kk + 2 < NFULL)

    scatter_wait(R[(NFULL - 2) % 3])
    scatter_wait(R[(NFULL - 1) % 3])

    tb = tbase + NFULL * CHM
    pltpu.sync_copy(src_hbm.at[pl.ds(tb, TL)], srcT)
    pltpu.sync_copy(dst_hbm.at[pl.ds(tb, TL)], dstT)
    pltpu.sync_copy(sca_hbm.at[pl.ds(tb, TL)], scaT)
    rowsT = rows2.at[pl.ds(0, TL), :]
    pltpu.async_copy(h_hbm.at[srcT], rowsT, semT).wait()
    for g in range(TL // 16):
        scv16 = scaT[pl.ds(g * 16, 16)]
        for l in range(16):
            e = g * 16 + l
            scv = jnp.full((16,), scv16[l], jnp.float32)
            for j in range(D // 16):
                sl = pl.ds(j * 16, 16)
                rows2[e, sl] = rows2[e, sl] * scv
    pltpu.sync_copy(rowsT, acc.at[dstT], add=True)

    plsc.subcore_barrier()

    @pl.when(s < 10)
    def _():
        pltpu.sync_copy(acc.at[pl.ds(s * 1000, 1000), :],
                        out_hbm.at[pl.ds(c * N + s * 1000, 1000), :])


@functools.cache
def _sc_kernels():
    mesh = plsc.VectorSubcoreMesh(core_axis_name="c", subcore_axis_name="s",
                                  num_cores=NC, num_subcores=NS)
    deg_kernel = pl.kernel(
        _deg_body,
        out_type=jax.ShapeDtypeStruct((NW * 3 * N,), jnp.float32),
        mesh=mesh,
        compiler_params=pltpu.CompilerParams(needs_layout_passes=False),
        scratch_types=[
            pltpu.VMEM((DBLK,), jnp.int32),
            pltpu.VMEM((DBLK,), jnp.int32),
            pltpu.VMEM((DBLK,), jnp.float32),
            pltpu.VMEM((DBLK,), jnp.int32),
            pltpu.VMEM((DBLK,), jnp.int32),
            pltpu.VMEM((DBLK,), jnp.float32),
            pltpu.VMEM((3 * N,), jnp.float32),
            pltpu.SemaphoreType.DMA,
            pltpu.SemaphoreType.DMA,
        ],
    )
    mp_kernel = pl.kernel(
        _mp_body,
        out_type=jax.ShapeDtypeStruct((NC * N, D), jnp.float32),
        mesh=mesh,
        scratch_types=[
            pltpu.VMEM((CHM,), jnp.int32),
            pltpu.VMEM((CHM,), jnp.int32),
            pltpu.VMEM((CHM,), jnp.float32),
            pltpu.VMEM((CHM,), jnp.int32),
            pltpu.VMEM((CHM,), jnp.int32),
            pltpu.VMEM((CHM,), jnp.float32),
            pltpu.VMEM((CHM,), jnp.int32),
            pltpu.VMEM((CHM,), jnp.int32),
            pltpu.VMEM((CHM,), jnp.int32),
            pltpu.VMEM((CHM, D), jnp.float32),
            pltpu.VMEM((CHM, D), jnp.float32),
            pltpu.VMEM((CHM, D), jnp.float32),
            pltpu.VMEM((TL,), jnp.int32),
            pltpu.VMEM((TL,), jnp.int32),
            pltpu.VMEM((TL,), jnp.float32),
            pltpu.VMEM_SHARED((N, D), jnp.float32),
            pltpu.SemaphoreType.DMA,
            pltpu.SemaphoreType.DMA,
            pltpu.SemaphoreType.DMA,
            pltpu.SemaphoreType.DMA,
            pltpu.SemaphoreType.DMA,
            pltpu.SemaphoreType.DMA,
            pltpu.SemaphoreType.DMA,
            pltpu.SemaphoreType.DMA,
            pltpu.SemaphoreType.DMA,
        ],
    )
    return deg_kernel, mp_kernel


def _tcA_body(x_ref, w1_ref, degT_ref, eat_ref, we_ref, h_out, gate_out):
    dsrc = jnp.sum(degT_ref[0:N, :], axis=1, keepdims=True)
    a = lax.rsqrt(jnp.clip(dsrc, 1e-12))
    h = jnp.dot(x_ref[...], w1_ref[...], preferred_element_type=jnp.float32)
    h_out[...] = h * a
    gate_out[...] = jnp.sum(eat_ref[...] * we_ref[...], axis=0)


def _tcB_body(p_ref, degT_ref, b1_ref, g1_ref, be1_ref, w2_ref, h2_out):
    p = p_ref[...]
    agg = p[:N] + p[N:]
    ddst = jnp.sum(degT_ref[N:2 * N, :], axis=1, keepdims=True)
    bfac = lax.rsqrt(jnp.clip(ddst, 1e-12))
    agg = agg * bfac + b1_ref[...]
    mean = jnp.mean(agg, axis=0, keepdims=True)
    var = jnp.mean((agg - mean) ** 2, axis=0, keepdims=True)
    hb = (agg - mean) * lax.rsqrt(var + 1e-5) * g1_ref[...] + be1_ref[...]
    h2_out[...] = jnp.dot(hb, w2_ref[...],
                          preferred_element_type=jnp.float32)


def _tcC_body(p_ref, degT_ref, b2_ref, g2_ref, be2_ref, out_ref):
    p = p_ref[...]
    agg = p[:N] + p[N:]
    cnt = jnp.sum(degT_ref[2 * N:3 * N, :], axis=1, keepdims=True)
    agg = agg / jnp.clip(cnt, 1.0) + b2_ref[...]
    mean = jnp.mean(agg, axis=0, keepdims=True)
    var = jnp.mean((agg - mean) ** 2, axis=0, keepdims=True)
    hb = (agg - mean) * lax.rsqrt(var + 1e-5) * g2_ref[...] + be2_ref[...]
    out_ref[...] = jnp.where(hb > 0, hb, 0.1 * (jnp.exp(hb) - 1.0))


def kernel(x, edge_index, edge_attrs, W1, b1, g1, be1, we, W2, b2, g2, be2):
    src = edge_index[0]
    dst = edge_index[1]
    ew = edge_attrs[:, 1]
    eat = edge_attrs.T.reshape(4, E // D, D)

    deg_kernel, mp_kernel = _sc_kernels()
    degs = deg_kernel(src, dst, ew)
    degT = degs.reshape(NW, 3 * N).T

    h1s, gate2d = pl.pallas_call(
        _tcA_body,
        out_shape=[jax.ShapeDtypeStruct((N, D), jnp.float32),
                   jax.ShapeDtypeStruct((E // D, D), jnp.float32)],
    )(x, W1, degT, eat, we.reshape(4, 1, 1))
    gate = gate2d.reshape(E)

    p1 = mp_kernel(src, dst, ew, h1s)

    h2 = pl.pallas_call(
        _tcB_body,
        out_shape=jax.ShapeDtypeStruct((N, D), jnp.float32),
    )(p1, degT, b1.reshape(1, D), g1.reshape(1, D), be1.reshape(1, D), W2)

    p2 = mp_kernel(src, dst, gate, h2)

    out = pl.pallas_call(
        _tcC_body,
        out_shape=jax.ShapeDtypeStruct((N, D), jnp.float32),
    )(p2, degT, b2.reshape(1, D), g2.reshape(1, D), be2.reshape(1, D))
    return out


# speedup vs baseline: 1.5924x; 1.0002x over previous
import functools

import jax
import jax.numpy as jnp
from jax import lax
from jax.experimental import pallas as pl
from jax.experimental.pallas import tpu as pltpu
from jax.experimental.pallas import tpu_sc as plsc

N = 10000
E = 320000
D = 128
NC = 2
NS = 16
NW = NC * NS
EPT = E // NW
CHM = 112
NFULL = 89
TL = EPT - NFULL * CHM
NIT = 14

DBLK = 2000
NDBLK = EPT // DBLK


def _deg_body(src_hbm, dst_hbm, ew_hbm, out_hbm, bsrc, bdst, bew,
              bsrc2, bdst2, bew2, accl, semA, semB):
    c = lax.axis_index("c")
    s = lax.axis_index("s")
    w = c * NS + s

    zero16 = jnp.zeros((16,), jnp.float32)
    one16 = jnp.ones((16,), jnp.float32)
    off16 = jnp.full((16,), N, jnp.int32)

    def zfill(i, _):
        accl[pl.ds(i * 16, 16)] = zero16
        return 0
    lax.fori_loop(0, (3 * N) // 16, zfill, 0)

    B = [(bsrc, bdst, bew, semA), (bsrc2, bdst2, bew2, semB)]

    def dblk_start(b, bb):
        base = w * EPT + b * DBLK
        pltpu.async_copy(src_hbm.at[pl.ds(base, DBLK)], bb[0], bb[3])
        pltpu.async_copy(dst_hbm.at[pl.ds(base, DBLK)], bb[1], bb[3])
        pltpu.async_copy(ew_hbm.at[pl.ds(base, DBLK)], bb[2], bb[3])

    def dblk_wait(b, bb):
        base = w * EPT + b * DBLK
        pltpu.make_async_copy(src_hbm.at[pl.ds(base, DBLK)], bb[0],
                              bb[3]).wait()
        pltpu.make_async_copy(dst_hbm.at[pl.ds(base, DBLK)], bb[1],
                              bb[3]).wait()
        pltpu.make_async_copy(ew_hbm.at[pl.ds(base, DBLK)], bb[2],
                              bb[3]).wait()

    def dblk_compute(bb):
        def grp(i, _):
            sl = pl.ds(i * 16, 16)
            s16 = bb[0][sl]
            d16 = bb[1][sl] + off16
            e16 = bb[2][sl]
            plsc.addupdate_scatter(accl, [s16], e16)
            plsc.addupdate_scatter(accl, [d16], e16)
            plsc.addupdate_scatter(accl, [d16 + off16], one16)
            return 0
        lax.fori_loop(0, DBLK // 16, grp, 0)

    dblk_start(0, B[0])
    for b in range(NDBLK):
        if b + 1 < NDBLK:
            dblk_start(b + 1, B[(b + 1) % 2])
        dblk_wait(b, B[b % 2])
        dblk_compute(B[b % 2])

    pltpu.sync_copy(accl, out_hbm.at[pl.ds(w * 3 * N, 3 * N)])


def _mp_body(src_hbm, dst_hbm, sca_hbm, h_hbm, out_hbm,
             src0, dst0, sca0, src1, dst1, sca1,
             d2_0, d2_1, d2_2, rows0, rows1, rows2,
             srcT, dstT, scaT, acc,
             semL0, semL1, semG0, semG1, semG2, semS0, semS1, semS2,
             semT):
    c = lax.axis_index("c")
    s = lax.axis_index("s")
    w = c * NS + s
    tbase = w * EPT

    I = [(src0, dst0, sca0, semL0), (src1, dst1, sca1, semL1)]
    R = [(rows0, d2_0, semG0, semS0), (rows1, d2_1, semG1, semS1),
         (rows2, d2_2, semG2, semS2)]

    zero16 = jnp.zeros((16,), jnp.float32)

    def zfill(i, _):
        for j in range(D // 16):
            rows0[i, pl.ds(j * 16, 16)] = zero16
        return 0
    lax.fori_loop(0, CHM, zfill, 0)

    @pl.when(s < 10)
    def _():
        for i in range(8):
            pltpu.async_copy(rows0,
                             acc.at[pl.ds(s * 1000 + i * CHM, CHM), :],
                             semL0)
        pltpu.async_copy(rows0.at[pl.ds(0, 104), :],
                         acc.at[pl.ds(s * 1000 + 896, 104), :], semL0)
        for i in range(8):
            pltpu.make_async_copy(
                rows0, acc.at[pl.ds(s * 1000 + i * CHM, CHM), :],
                semL0).wait()
        pltpu.make_async_copy(rows0.at[pl.ds(0, 104), :],
                              acc.at[pl.ds(s * 1000 + 896, 104), :],
                              semL0).wait()
    plsc.subcore_barrier()

    def loads_start(k, ib):
        s_v, d_v, e_v, sem = ib
        base = tbase + k * CHM
        pltpu.async_copy(src_hbm.at[pl.ds(base, CHM)], s_v, sem)
        pltpu.async_copy(dst_hbm.at[pl.ds(base, CHM)], d_v, sem)
        pltpu.async_copy(sca_hbm.at[pl.ds(base, CHM)], e_v, sem)

    def loads_wait(k, ib):
        s_v, d_v, e_v, sem = ib
        base = tbase + k * CHM
        pltpu.make_async_copy(src_hbm.at[pl.ds(base, CHM)], s_v, sem).wait()
        pltpu.make_async_copy(dst_hbm.at[pl.ds(base, CHM)], d_v, sem).wait()
        pltpu.make_async_copy(sca_hbm.at[pl.ds(base, CHM)], e_v, sem).wait()

    def idx_copy(ib, rb):
        d_v = ib[1]
        d2_v = rb[1]
        for j in range(CHM // 16):
            sl = pl.ds(j * 16, 16)
            d2_v[sl] = d_v[sl]

    def gather_start(ib, rb):
        pltpu.async_copy(h_hbm.at[ib[0]], rb[0], rb[2])

    def gather_wait(ib, rb):
        pltpu.make_async_copy(h_hbm.at[ib[0]], rb[0], rb[2]).wait()

    def scatter_start(rb):
        pltpu.async_copy(rb[0], acc.at[rb[1]], rb[3], add=True)

    def scatter_wait(rb):
        pltpu.make_async_copy(rb[0], acc.at[rb[1]], rb[3]).wait()

    def compute_chunk(rb, ib):
        rows = rb[0]
        e_v = ib[2]

        def erow(g, _):
            scv16 = e_v[pl.ds(g * 16, 16)]
            for l in range(16):
                e = g * 16 + l
                scv = jnp.full((16,), scv16[l], jnp.float32)
                for j in range(D // 16):
                    sl = pl.ds(j * 16, 16)
                    rows[e, sl] = rows[e, sl] * scv
            return 0
        lax.fori_loop(0, CHM // 16, erow, 0)

    def step(k, u, it, guard_first):
        icur = I[u % 2]
        inext = I[(u + 1) % 2]
        rcur = R[u % 3]
        rnext = R[(u + 1) % 3]
        loads_wait(k + 1, inext)
        if guard_first:
            @pl.when(it > 0)
            def _():
                scatter_wait(rnext)
        else:
            scatter_wait(rnext)
        idx_copy(inext, rnext)
        gather_start(inext, rnext)
        gather_wait(icur, rcur)
        compute_chunk(rcur, icur)
        scatter_start(rcur)
        loads_start(k + 2, icur)

    def tail_step(kk, u, has_next, has_next2):
        icur = I[u % 2]
        inext = I[(u + 1) % 2]
        rcur = R[u % 3]
        rnext = R[(u + 1) % 3]
        if has_next:
            loads_wait(kk + 1, inext)
        scatter_wait(rnext)
        if has_next:
            idx_copy(inext, rnext)
            gather_start(inext, rnext)
        gather_wait(icur, rcur)
        compute_chunk(rcur, icur)
        scatter_start(rcur)
        if has_next2:
            loads_start(kk + 2, icur)

    loads_start(0, I[0])
    loads_wait(0, I[0])
    idx_copy(I[0], R[0])
    gather_start(I[0], R[0])
    loads_start(1, I[1])

    def six(it, _):
        for u in range(6):
            step(6 * it + u, u, it, u < 2)
        return 0
    lax.fori_loop(0, NIT, six, 0)

    for kk in range(6 * NIT, NFULL):
        tail_step(kk, kk % 6, kk + 1 < NFULL, kk + 2 < NFULL)

    scatter_wait(R[(NFULL - 2) % 3])
    scatter_wait(R[(NFULL - 1) % 3])

    tb = tbase + NFULL * CHM
    pltpu.sync_copy(src_hbm.at[pl.ds(tb, TL)], srcT)
    pltpu.sync_copy(dst_hbm.at[pl.ds(tb, TL)], dstT)
    pltpu.sync_copy(sca_hbm.at[pl.ds(tb, TL)], scaT)
    rowsT = rows2.at[pl.ds(0, TL), :]
    pltpu.async_copy(h_hbm.at[srcT], rowsT, semT).wait()
    for g in range(TL // 16):
        scv16 = scaT[pl.ds(g * 16, 16)]
        for l in range(16):
            e = g * 16 + l
            scv = jnp.full((16,), scv16[l], jnp.float32)
            for j in range(D // 16):
                sl = pl.ds(j * 16, 16)
                rows2[e, sl] = rows2[e, sl] * scv
    pltpu.sync_copy(rowsT, acc.at[dstT], add=True)

    plsc.subcore_barrier()

    @pl.when(s < 10)
    def _():
        pltpu.sync_copy(acc.at[pl.ds(s * 1000, 1000), :],
                        out_hbm.at[pl.ds(c * N + s * 1000, 1000), :])


@functools.cache
def _sc_kernels():
    mesh = plsc.VectorSubcoreMesh(core_axis_name="c", subcore_axis_name="s",
                                  num_cores=NC, num_subcores=NS)
    deg_kernel = pl.kernel(
        _deg_body,
        out_type=jax.ShapeDtypeStruct((NW * 3 * N,), jnp.float32),
        mesh=mesh,
        compiler_params=pltpu.CompilerParams(needs_layout_passes=False),
        scratch_types=[
            pltpu.VMEM((DBLK,), jnp.int32),
            pltpu.VMEM((DBLK,), jnp.int32),
            pltpu.VMEM((DBLK,), jnp.float32),
            pltpu.VMEM((DBLK,), jnp.int32),
            pltpu.VMEM((DBLK,), jnp.int32),
            pltpu.VMEM((DBLK,), jnp.float32),
            pltpu.VMEM((3 * N,), jnp.float32),
            pltpu.SemaphoreType.DMA,
            pltpu.SemaphoreType.DMA,
        ],
    )
    mp_kernel = pl.kernel(
        _mp_body,
        out_type=jax.ShapeDtypeStruct((NC * N, D), jnp.float32),
        mesh=mesh,
        scratch_types=[
            pltpu.VMEM((CHM,), jnp.int32),
            pltpu.VMEM((CHM,), jnp.int32),
            pltpu.VMEM((CHM,), jnp.float32),
            pltpu.VMEM((CHM,), jnp.int32),
            pltpu.VMEM((CHM,), jnp.int32),
            pltpu.VMEM((CHM,), jnp.float32),
            pltpu.VMEM((CHM,), jnp.int32),
            pltpu.VMEM((CHM,), jnp.int32),
            pltpu.VMEM((CHM,), jnp.int32),
            pltpu.VMEM((CHM, D), jnp.float32),
            pltpu.VMEM((CHM, D), jnp.float32),
            pltpu.VMEM((CHM, D), jnp.float32),
            pltpu.VMEM((TL,), jnp.int32),
            pltpu.VMEM((TL,), jnp.int32),
            pltpu.VMEM((TL,), jnp.float32),
            pltpu.VMEM_SHARED((N, D), jnp.float32),
            pltpu.SemaphoreType.DMA,
            pltpu.SemaphoreType.DMA,
            pltpu.SemaphoreType.DMA,
            pltpu.SemaphoreType.DMA,
            pltpu.SemaphoreType.DMA,
            pltpu.SemaphoreType.DMA,
            pltpu.SemaphoreType.DMA,
            pltpu.SemaphoreType.DMA,
            pltpu.SemaphoreType.DMA,
        ],
    )
    return deg_kernel, mp_kernel


def _tcA_body(x_ref, w1_ref, degT_ref, eat_ref, we_ref, h_out, gate_out):
    dsrc = jnp.sum(degT_ref[0:N, :], axis=1, keepdims=True)
    a = lax.rsqrt(jnp.clip(dsrc, 1e-12))
    h = jnp.dot(x_ref[...], w1_ref[...], preferred_element_type=jnp.float32)
    h_out[...] = h * a
    gate_out[...] = jnp.sum(eat_ref[...] * we_ref[...], axis=0)


def _tcB_body(p_ref, degT_ref, b1_ref, g1_ref, be1_ref, w2_ref, h2_out):
    p = p_ref[...]
    agg = p[:N] + p[N:]
    ddst = jnp.sum(degT_ref[N:2 * N, :], axis=1, keepdims=True)
    bfac = lax.rsqrt(jnp.clip(ddst, 1e-12))
    agg = agg * bfac + b1_ref[...]
    mean = jnp.mean(agg, axis=0, keepdims=True)
    var = jnp.mean((agg - mean) ** 2, axis=0, keepdims=True)
    hb = (agg - mean) * lax.rsqrt(var + 1e-5) * g1_ref[...] + be1_ref[...]
    h2_out[...] = jnp.dot(hb, w2_ref[...],
                          preferred_element_type=jnp.float32)


def _tcC_body(p_ref, degT_ref, b2_ref, g2_ref, be2_ref, out_ref):
    p = p_ref[...]
    agg = p[:N] + p[N:]
    cnt = jnp.sum(degT_ref[2 * N:3 * N, :], axis=1, keepdims=True)
    agg = agg / jnp.clip(cnt, 1.0) + b2_ref[...]
    mean = jnp.mean(agg, axis=0, keepdims=True)
    var = jnp.mean((agg - mean) ** 2, axis=0, keepdims=True)
    hb = (agg - mean) * lax.rsqrt(var + 1e-5) * g2_ref[...] + be2_ref[...]
    out_ref[...] = jnp.where(hb > 0, hb, 0.1 * (jnp.exp(hb) - 1.0))


def kernel(x, edge_index, edge_attrs, W1, b1, g1, be1, we, W2, b2, g2, be2):
    src = edge_index[0]
    dst = edge_index[1]
    ew = edge_attrs[:, 1]
    eat = edge_attrs.T.reshape(4, E // D, D)

    deg_kernel, mp_kernel = _sc_kernels()
    degs = deg_kernel(src, dst, ew)
    degT = degs.reshape(NW, 3 * N).T

    h1s, gate2d = pl.pallas_call(
        _tcA_body,
        out_shape=[jax.ShapeDtypeStruct((N, D), jnp.float32),
                   jax.ShapeDtypeStruct((E // D, D), jnp.float32)],
    )(x, W1, degT, eat, we.reshape(4, 1, 1))
    gate = gate2d.reshape(E)

    p1 = mp_kernel(src, dst, ew, h1s)

    h2 = pl.pallas_call(
        _tcB_body,
        out_shape=jax.ShapeDtypeStruct((N, D), jnp.float32),
    )(p1, degT, b1.reshape(1, D), g1.reshape(1, D), be1.reshape(1, D), W2)

    p2 = mp_kernel(src, dst, gate, h2)

    out = pl.pallas_call(
        _tcC_body,
        out_shape=jax.ShapeDtypeStruct((N, D), jnp.float32),
    )(p2, degT, b2.reshape(1, D), g2.reshape(1, D), be2.reshape(1, D))
    return out
